# Initial kernel scaffold; baseline (speedup 1.0000x reference)
import functools

import jax
import jax.numpy as jnp
from jax import lax
from jax.experimental import pallas as pl
from jax.experimental.pallas import tpu as pltpu
from jax.experimental.pallas import tpu_sc as plsc

NUM_ENTITIES = 1000000
NUM_RELATIONS = 1000
HALF_DIM = 64
ROW = 2 * HALF_DIM
BATCH = 16384

_NC = 2
_NS = 16
_NW = _NC * _NS
_PER_W = BATCH // _NW
_CHUNK = 128
_NCHUNK = _PER_W // _CHUNK


def _trig_body(r_ref, cos_ref, sin_ref):
    r = r_ref[...]
    cos_ref[...] = jnp.cos(r)
    sin_ref[...] = jnp.sin(r)


def _trig_tables(relation_emb):
    return pl.pallas_call(
        _trig_body,
        out_shape=[
            jax.ShapeDtypeStruct((NUM_RELATIONS, HALF_DIM), jnp.float32),
            jax.ShapeDtypeStruct((NUM_RELATIONS, HALF_DIM), jnp.float32),
        ],
    )(relation_emb)


def _newton_sqrt(x):
    xc = jnp.maximum(x, jnp.float32(1e-30))
    i = jnp.int32(0x5F3759DF) - (plsc.bitcast(xc, jnp.int32) >> 1)
    y = plsc.bitcast(i, jnp.float32)
    half = jnp.float32(0.5) * xc
    for _ in range(3):
        y = y * (jnp.float32(1.5) - half * y * y)
    return xc * y


def _sc_body(heads_hbm, rels_hbm, tails_hbm, entity_hbm, cos_hbm, sin_hbm,
             out_hbm, hidx_v, ridx_v, tidx_v, h_v, t_v, c_v, s_v, out_v, sem):
    wid = lax.axis_index("s") * _NC + lax.axis_index("c")
    for chunk in range(_NCHUNK):
        base = wid * _PER_W + chunk * _CHUNK
        pltpu.sync_copy(heads_hbm.at[pl.ds(base, _CHUNK)], hidx_v)
        pltpu.sync_copy(rels_hbm.at[pl.ds(base, _CHUNK)], ridx_v)
        pltpu.sync_copy(tails_hbm.at[pl.ds(base, _CHUNK)], tidx_v)
        cp_h = pltpu.async_copy(entity_hbm.at[hidx_v], h_v, sem)
        cp_t = pltpu.async_copy(entity_hbm.at[tidx_v], t_v, sem)
        cp_c = pltpu.async_copy(cos_hbm.at[ridx_v], c_v, sem)
        cp_s = pltpu.async_copy(sin_hbm.at[ridx_v], s_v, sem)
        cp_h.wait()
        cp_t.wait()
        cp_c.wait()
        cp_s.wait()

        def item_body(i, carry):
            acc = jnp.zeros((16,), jnp.float32)
            for g in range(HALF_DIM // 16):
                re_sl = pl.ds(g * 16, 16)
                im_sl = pl.ds(HALF_DIM + g * 16, 16)
                hre = h_v[i, re_sl]
                him = h_v[i, im_sl]
                tre = t_v[i, re_sl]
                tim = t_v[i, im_sl]
                c = c_v[i, re_sl]
                s = s_v[i, re_sl]
                hr_re = hre * c - him * s
                hr_im = hre * s + him * c
                dre = hr_re - tre
                dim_ = hr_im - tim
                acc = acc + _newton_sqrt(dre * dre + dim_ * dim_)
            out_v[i] = jnp.sum(acc)
            return carry

        lax.fori_loop(0, _CHUNK, item_body, 0)
        pltpu.sync_copy(out_v, out_hbm.at[pl.ds(base, _CHUNK)])


@jax.jit
def _rotate_score(heads, rels, tails, entity_emb, cos_t, sin_t):
    mesh = plsc.VectorSubcoreMesh(core_axis_name="c", subcore_axis_name="s")
    kfn = pl.kernel(
        _sc_body,
        out_type=jax.ShapeDtypeStruct((BATCH,), jnp.float32),
        mesh=mesh,
        scratch_types=[
            pltpu.VMEM((_CHUNK,), jnp.int32),
            pltpu.VMEM((_CHUNK,), jnp.int32),
            pltpu.VMEM((_CHUNK,), jnp.int32),
            pltpu.VMEM((_CHUNK, ROW), jnp.float32),
            pltpu.VMEM((_CHUNK, ROW), jnp.float32),
            pltpu.VMEM((_CHUNK, HALF_DIM), jnp.float32),
            pltpu.VMEM((_CHUNK, HALF_DIM), jnp.float32),
            pltpu.VMEM((_CHUNK,), jnp.float32),
            pltpu.SemaphoreType.DMA,
        ],
    )
    return kfn(heads, rels, tails, entity_emb, cos_t, sin_t)


def kernel(heads, relations, tails, entity_emb, relation_emb):
    heads = heads.astype(jnp.int32)
    relations = relations.astype(jnp.int32)
    tails = tails.astype(jnp.int32)
    cos_t, sin_t = _trig_tables(relation_emb)
    return _rotate_score(heads, relations, tails, entity_emb, cos_t, sin_t)


# speedup vs baseline: 2.4419x; 2.4419x over previous
import functools

import jax
import jax.numpy as jnp
from jax import lax
from jax.experimental import pallas as pl
from jax.experimental.pallas import tpu as pltpu
from jax.experimental.pallas import tpu_sc as plsc

NUM_ENTITIES = 1000000
NUM_RELATIONS = 1000
HALF_DIM = 64
ROW = 2 * HALF_DIM
BATCH = 16384

_NC = 2
_NS = 16
_NW = _NC * _NS
_PER_W = BATCH // _NW
_CHUNK = 128
_NCHUNK = _PER_W // _CHUNK


def _trig_body(r_ref, trig_ref):
    r = r_ref[...]
    trig_ref[...] = jnp.concatenate([jnp.cos(r), jnp.sin(r)], axis=-1)


def _trig_tables(relation_emb):
    return pl.pallas_call(
        _trig_body,
        out_shape=jax.ShapeDtypeStruct((NUM_RELATIONS, ROW), jnp.float32),
    )(relation_emb)


def _newton_sqrt(x):
    xc = jnp.maximum(x, jnp.float32(1e-30))
    i = jnp.int32(0x5F3759DF) - (lax.bitcast_convert_type(xc, jnp.int32) >> 1)
    y = lax.bitcast_convert_type(i, jnp.float32)
    half = jnp.float32(0.5) * xc
    for _ in range(3):
        y = y * (jnp.float32(1.5) - half * y * y)
    return xc * y


def _sc_body(heads_hbm, rels_hbm, tails_hbm, entity_hbm, trig_hbm,
             out_hbm, hidx_v, ridx_v, tidx_v, h_v, t_v, trig_v, out_v, sem):
    wid = lax.axis_index("s") * _NC + lax.axis_index("c")
    for chunk in range(_NCHUNK):
        base = wid * _PER_W + chunk * _CHUNK
        pltpu.sync_copy(heads_hbm.at[pl.ds(base, _CHUNK)], hidx_v)
        pltpu.sync_copy(rels_hbm.at[pl.ds(base, _CHUNK)], ridx_v)
        pltpu.sync_copy(tails_hbm.at[pl.ds(base, _CHUNK)], tidx_v)
        cp_h = pltpu.async_copy(entity_hbm.at[hidx_v], h_v, sem)
        cp_t = pltpu.async_copy(entity_hbm.at[tidx_v], t_v, sem)
        cp_r = pltpu.async_copy(trig_hbm.at[ridx_v], trig_v, sem)
        cp_h.wait()
        cp_t.wait()
        cp_r.wait()

        lane = lax.iota(jnp.int32, 16)

        def group_body(gi, carry):
            score_vec = jnp.zeros((16,), jnp.float32)
            for k in range(16):
                i = gi * 16 + k
                acc = jnp.zeros((16,), jnp.float32)
                for g in range(HALF_DIM // 16):
                    re_sl = pl.ds(g * 16, 16)
                    im_sl = pl.ds(HALF_DIM + g * 16, 16)
                    hre = h_v[i, re_sl]
                    him = h_v[i, im_sl]
                    tre = t_v[i, re_sl]
                    tim = t_v[i, im_sl]
                    c = trig_v[i, re_sl]
                    s = trig_v[i, im_sl]
                    hr_re = hre * c - him * s
                    hr_im = hre * s + him * c
                    dre = hr_re - tre
                    dim_ = hr_im - tim
                    acc = acc + _newton_sqrt(dre * dre + dim_ * dim_)
                score_vec = jnp.where(lane == k, jnp.sum(acc), score_vec)
            out_v[pl.ds(gi * 16, 16)] = score_vec
            return carry

        lax.fori_loop(0, _CHUNK // 16, group_body, 0)
        pltpu.sync_copy(out_v, out_hbm.at[pl.ds(base, _CHUNK)])


@jax.jit
def _rotate_score(heads, rels, tails, entity_emb, trig_t):
    mesh = plsc.VectorSubcoreMesh(core_axis_name="c", subcore_axis_name="s")
    kfn = pl.kernel(
        _sc_body,
        out_type=jax.ShapeDtypeStruct((BATCH,), jnp.float32),
        mesh=mesh,
        compiler_params=pltpu.CompilerParams(needs_layout_passes=False),
        scratch_types=[
            pltpu.VMEM((_CHUNK,), jnp.int32),
            pltpu.VMEM((_CHUNK,), jnp.int32),
            pltpu.VMEM((_CHUNK,), jnp.int32),
            pltpu.VMEM((_CHUNK, ROW), jnp.float32),
            pltpu.VMEM((_CHUNK, ROW), jnp.float32),
            pltpu.VMEM((_CHUNK, ROW), jnp.float32),
            pltpu.VMEM((_CHUNK,), jnp.float32),
            pltpu.SemaphoreType.DMA,
        ],
    )
    return kfn(heads, rels, tails, entity_emb, trig_t)


def kernel(heads, relations, tails, entity_emb, relation_emb):
    heads = heads.astype(jnp.int32)
    relations = relations.astype(jnp.int32)
    tails = tails.astype(jnp.int32)
    trig_t = _trig_tables(relation_emb)
    return _rotate_score(heads, relations, tails, entity_emb, trig_t)


# speedup vs baseline: 3.1523x; 1.2909x over previous
import jax
import jax.numpy as jnp
from jax import lax
from jax.experimental import pallas as pl
from jax.experimental.pallas import tpu as pltpu
from jax.experimental.pallas import tpu_sc as plsc

NUM_ENTITIES = 1000000
NUM_RELATIONS = 1000
HALF_DIM = 64
ROW = 2 * HALF_DIM
BATCH = 16384

_NC = 2
_NS = 16
_NW = _NC * _NS
_PER_W = BATCH // _NW
_CHUNK = 128
_NCHUNK = _PER_W // _CHUNK


def _trig_body(r_ref, trig_ref):
    r = r_ref[...]
    trig_ref[...] = jnp.concatenate([jnp.cos(r), jnp.sin(r)], axis=-1)


def _trig_tables(relation_emb):
    return pl.pallas_call(
        _trig_body,
        out_shape=jax.ShapeDtypeStruct((NUM_RELATIONS, ROW), jnp.float32),
    )(relation_emb)


def _newton_sqrt(x):
    xc = jnp.maximum(x, jnp.float32(1e-30))
    i = jnp.int32(0x5F3759DF) - (lax.bitcast_convert_type(xc, jnp.int32) >> 1)
    y = lax.bitcast_convert_type(i, jnp.float32)
    half = jnp.float32(0.5) * xc
    for _ in range(2):
        y = y * (jnp.float32(1.5) - half * y * y)
    return xc * y


def _sc_body(heads_hbm, rels_hbm, tails_hbm, entity_hbm, trig_hbm,
             out_hbm, hidx_v, ridx_v, tidx_v, h_v, t_v, trig_v, out_v,
             gsem0, gsem1, osem0, osem1):
    wid = lax.axis_index("s") * _NC + lax.axis_index("c")
    base_w = wid * _PER_W
    pltpu.sync_copy(heads_hbm.at[pl.ds(base_w, _PER_W)], hidx_v)
    pltpu.sync_copy(rels_hbm.at[pl.ds(base_w, _PER_W)], ridx_v)
    pltpu.sync_copy(tails_hbm.at[pl.ds(base_w, _PER_W)], tidx_v)

    gsems = (gsem0, gsem1)
    osems = (osem0, osem1)

    def issue(c):
        b = c & 1
        csl = pl.ds(c * _CHUNK, _CHUNK)
        return (
            pltpu.async_copy(entity_hbm.at[hidx_v.at[csl]], h_v.at[b], gsems[b]),
            pltpu.async_copy(entity_hbm.at[tidx_v.at[csl]], t_v.at[b], gsems[b]),
            pltpu.async_copy(trig_hbm.at[ridx_v.at[csl]], trig_v.at[b], gsems[b]),
        )

    lane = lax.iota(jnp.int32, 16)
    pending = {0: issue(0)}
    out_cps = {}
    for c in range(_NCHUNK):
        b = c & 1
        if c + 1 < _NCHUNK:
            pending[c + 1] = issue(c + 1)
        for cp in pending.pop(c):
            cp.wait()
        if c >= 2:
            out_cps.pop(c - 2).wait()

        def group_body(gi, carry):
            score_vec = jnp.zeros((16,), jnp.float32)
            for k in range(16):
                i = gi * 16 + k
                acc = jnp.zeros((16,), jnp.float32)
                for g in range(HALF_DIM // 16):
                    re_sl = pl.ds(g * 16, 16)
                    im_sl = pl.ds(HALF_DIM + g * 16, 16)
                    hre = h_v[b, i, re_sl]
                    him = h_v[b, i, im_sl]
                    tre = t_v[b, i, re_sl]
                    tim = t_v[b, i, im_sl]
                    co = trig_v[b, i, re_sl]
                    si = trig_v[b, i, im_sl]
                    hr_re = hre * co - him * si
                    hr_im = hre * si + him * co
                    dre = hr_re - tre
                    dim_ = hr_im - tim
                    acc = acc + _newton_sqrt(dre * dre + dim_ * dim_)
                score_vec = jnp.where(lane == k, jnp.sum(acc), score_vec)
            out_v[b, pl.ds(gi * 16, 16)] = score_vec
            return carry

        lax.fori_loop(0, _CHUNK // 16, group_body, 0)
        out_cps[c] = pltpu.async_copy(
            out_v.at[b], out_hbm.at[pl.ds(base_w + c * _CHUNK, _CHUNK)], osems[b])
    for c in sorted(out_cps):
        out_cps[c].wait()


@jax.jit
def _rotate_score(heads, rels, tails, entity_emb, trig_t):
    mesh = plsc.VectorSubcoreMesh(core_axis_name="c", subcore_axis_name="s")
    kfn = pl.kernel(
        _sc_body,
        out_type=jax.ShapeDtypeStruct((BATCH,), jnp.float32),
        mesh=mesh,
        compiler_params=pltpu.CompilerParams(needs_layout_passes=False),
        scratch_types=[
            pltpu.VMEM((_PER_W,), jnp.int32),
            pltpu.VMEM((_PER_W,), jnp.int32),
            pltpu.VMEM((_PER_W,), jnp.int32),
            pltpu.VMEM((2, _CHUNK, ROW), jnp.float32),
            pltpu.VMEM((2, _CHUNK, ROW), jnp.float32),
            pltpu.VMEM((2, _CHUNK, ROW), jnp.float32),
            pltpu.VMEM((2, _CHUNK), jnp.float32),
            pltpu.SemaphoreType.DMA,
            pltpu.SemaphoreType.DMA,
            pltpu.SemaphoreType.DMA,
            pltpu.SemaphoreType.DMA,
        ],
    )
    return kfn(heads, rels, tails, entity_emb, trig_t)


def kernel(heads, relations, tails, entity_emb, relation_emb):
    heads = heads.astype(jnp.int32)
    relations = relations.astype(jnp.int32)
    tails = tails.astype(jnp.int32)
    trig_t = _trig_tables(relation_emb)
    return _rotate_score(heads, relations, tails, entity_emb, trig_t)
